# baseline (speedup 1.0000x reference)
import functools

import jax
import jax.numpy as jnp
from jax import lax
from jax.experimental import pallas as pl
from jax.experimental.pallas import tpu as pltpu
from jax.experimental.pallas import tpu_sc as plsc

N = 10000
E = 160000
D_IN = 256
H = 512
B = 64
EXP_D = 128

N_PAD = 10240
E_PAD = 165888
E_PAD_DEG = 163840
NC = 2
NS = 16
KB = 128
NBAT = E_PAD // NS // KB
ROWS_PER_TILE = N_PAD // NS
ACC_ROWS = 10112
ACC_RPT = ACC_ROWS // NS
DUMMY = 10100
NBLK = 10
BLK = N_PAD // NBLK

_BN_C = float(1.0 / (1.0 + 1e-5) ** 0.5)


def _deg_body(dst_hbm, const_hbm, out_hbm, idx_v, ones_v, stage_v, acc_sh,
              sem):
    cid = lax.axis_index("c")
    sid = lax.axis_index("s")

    pltpu.sync_copy(const_hbm.at[0], ones_v)
    pltpu.sync_copy(const_hbm.at[1], stage_v)

    for j in range(ROWS_PER_TILE // KB):
        pltpu.sync_copy(
            stage_v, acc_sh.at[pl.ds(sid * ROWS_PER_TILE + j * KB, KB)])
    plsc.subcore_barrier()

    half = E_PAD_DEG // NC
    base = cid * half + sid * (half // NS)
    nbatch = (half // NS) // KB

    def body(j, _):
        pltpu.sync_copy(dst_hbm.at[pl.ds(base + j * KB, KB)], idx_v)
        pltpu.sync_copy(ones_v, acc_sh.at[idx_v], add=True)
        return 0
    lax.fori_loop(0, nbatch, body, 0)
    plsc.subcore_barrier()

    for cc in range(NC):
        @pl.when(cid == cc)
        def _():
            for j in range(ROWS_PER_TILE // KB):
                r0 = sid * ROWS_PER_TILE + j * KB
                pltpu.sync_copy(acc_sh.at[pl.ds(r0, KB)], stage_v)
                pltpu.sync_copy(stage_v, out_hbm.at[cc].at[pl.ds(r0, KB)])


_DEG_W = 128


def _agg_body(h_hbm, src_hbm, dst_hbm, out_hbm, src_ring, dst_ring, rows_v,
              acc_sh, gsem0, gsem1, gsem2, ssem0, ssem1, ssem2,
              xsem0, xsem1, xsem2, dsem0, dsem1, dsem2):
    cid = lax.axis_index("c")
    sid = lax.axis_index("s")

    gsems = (gsem0, gsem1, gsem2)
    ssems = (ssem0, ssem1, ssem2)
    xsems = (xsem0, xsem1, xsem2)
    dsems = (dsem0, dsem1, dsem2)

    def drain(dst_slot, sem):
        pltpu.make_async_copy(h_hbm.at[0].at[pl.ds(0, KB)], dst_slot,
                              sem).wait()

    def drain_idx(ring_slot, sem):
        pltpu.make_async_copy(src_hbm.at[0].at[0], ring_slot, sem).wait()

    for c in range(4):
        @pl.when(cid == (c // 2))
        def _():
            r0 = sid * ACC_RPT
            pltpu.sync_copy(h_hbm.at[c].at[pl.ds(r0, ACC_RPT)],
                            acc_sh.at[pl.ds(r0, ACC_RPT)])
        plsc.subcore_barrier()

        @pl.when(cid == (c // 2))
        def _():
            pltpu.async_copy(src_hbm.at[sid].at[0], src_ring.at[0],
                             xsems[0])
            pltpu.async_copy(src_hbm.at[sid].at[1], src_ring.at[1],
                             xsems[1])
            pltpu.async_copy(dst_hbm.at[sid].at[0], dst_ring.at[0],
                             dsems[0])
            drain_idx(src_ring.at[0], xsems[0])
            pltpu.async_copy(h_hbm.at[c].at[src_ring.at[0]], rows_v.at[0],
                             gsems[0])

            def group(gg, _):
                for b in range(3):
                    j = 3 * gg + b
                    s1 = (b + 1) % 3
                    s2 = (b + 2) % 3

                    @pl.when(j >= 2)
                    def _():
                        drain(rows_v.at[s1], ssems[s1])

                    @pl.when(j + 2 < NBAT)
                    def _():
                        pltpu.async_copy(src_hbm.at[sid].at[j + 2],
                                         src_ring.at[s2], xsems[s2])

                    @pl.when(j + 1 < NBAT)
                    def _():
                        pltpu.async_copy(dst_hbm.at[sid].at[j + 1],
                                         dst_ring.at[s1], dsems[s1])

                    @pl.when(j + 1 < NBAT)
                    def _():
                        drain_idx(src_ring.at[s1], xsems[s1])
                        pltpu.async_copy(h_hbm.at[c].at[src_ring.at[s1]],
                                         rows_v.at[s1], gsems[s1])

                    drain(rows_v.at[b], gsems[b])
                    drain_idx(dst_ring.at[b], dsems[b])
                    pltpu.async_copy(rows_v.at[b],
                                     acc_sh.at[dst_ring.at[b]],
                                     ssems[b], add=True)
                return 0
            lax.fori_loop(0, NBAT // 3, group, 0)
            drain(rows_v.at[(NBAT - 2) % 3], ssems[(NBAT - 2) % 3])
            drain(rows_v.at[(NBAT - 1) % 3], ssems[(NBAT - 1) % 3])
        plsc.subcore_barrier()

        @pl.when(cid == (c // 2))
        def _():
            r0 = sid * ACC_RPT
            pltpu.sync_copy(acc_sh.at[pl.ds(r0, ACC_RPT)],
                            out_hbm.at[c].at[pl.ds(r0, ACC_RPT)])
        plsc.subcore_barrier()


@functools.lru_cache(maxsize=None)
def _sc_kernels():
    mesh = plsc.VectorSubcoreMesh(core_axis_name="c", subcore_axis_name="s")
    deg = functools.partial(
        pl.kernel,
        mesh=mesh,
        out_type=jax.ShapeDtypeStruct((NC, N_PAD, _DEG_W), jnp.float32),
        scratch_types=[
            pltpu.VMEM((KB,), jnp.int32),
            pltpu.VMEM((KB, _DEG_W), jnp.float32),
            pltpu.VMEM((KB, _DEG_W), jnp.float32),
            pltpu.VMEM_SHARED((N_PAD, _DEG_W), jnp.float32),
            pltpu.SemaphoreType.DMA,
        ],
    )(_deg_body)
    agg = functools.partial(
        pl.kernel,
        mesh=mesh,
        out_type=jax.ShapeDtypeStruct((4, N_PAD, 128), jnp.float32),
        scratch_types=[
            pltpu.VMEM((3, KB), jnp.int32),
            pltpu.VMEM((3, KB), jnp.int32),
            pltpu.VMEM((3, KB, 128), jnp.float32),
            pltpu.VMEM_SHARED((ACC_ROWS, 128), jnp.float32),
        ] + [pltpu.SemaphoreType.DMA] * 12,
    )(_agg_body)
    return deg, agg


def _deg_kernel(dst):
    const = jnp.stack([jnp.ones((KB, _DEG_W), jnp.float32),
                       jnp.zeros((KB, _DEG_W), jnp.float32)])
    return _sc_kernels()[0](dst, const)


def _agg_kernel(h, src3, dst3):
    return _sc_kernels()[1](h, src3, dst3)


def _dinv(deg):
    return jnp.where(deg > 0.0, lax.rsqrt(deg), 0.0)


def _dot(a, b):
    return jnp.dot(a, b, preferred_element_type=jnp.float32,
                   precision=lax.Precision.HIGHEST)


def _tc1_body(x_ref, deg_ref, w_ref, out_ref):
    di = _dinv(deg_ref[...])
    h = _dot(x_ref[...], w_ref[...]) * di
    for c in range(4):
        out_ref[c] = h[:, c * 128:(c + 1) * 128]


def _tcmid_body(s_ref, deg_ref, al_ref, be_ref, w_ref, out_ref):
    di = _dinv(deg_ref[...])
    xb = jnp.concatenate([s_ref[c] for c in range(4)], axis=1)
    a = jnp.maximum(al_ref[...] * (xb * di) + be_ref[...], 0.0)
    h = _dot(a, w_ref[...]) * di
    for c in range(4):
        out_ref[c] = h[:, c * 128:(c + 1) * 128]


def _tcfinal_body(s_ref, deg_ref, al_ref, be_ref, brow_ref, bcol_ref,
                  bsmem_ref, ef_ref, we1_ref, bexp1_ref, we2_ref, bexp2_ref,
                  wf1_ref, bf1_ref, wf2_ref, bf2_ref,
                  out_ref, ge_ref, ee_ref,
                  mean_acc, max_acc, cnt_acc):
    i = pl.program_id(0)

    @pl.when(i == 0)
    def _():
        mean_acc[...] = jnp.zeros((B, H), jnp.float32)
        max_acc[...] = jnp.full((B, H), -3e38, jnp.float32)
        cnt_acc[...] = jnp.zeros((B, 1), jnp.float32)

    di = _dinv(deg_ref[...])
    xb = jnp.concatenate([s_ref[c] for c in range(4)], axis=1)
    h4 = jnp.maximum(al_ref[...] * (xb * di) + be_ref[...], 0.0)
    bcol = bcol_ref[...]
    h4 = jnp.where(bcol < B, h4, 0.0)

    brow = brow_ref[0]
    seg = lax.broadcasted_iota(jnp.int32, (B, BLK), 0)
    mask = (brow == seg).astype(jnp.float32)
    mean_acc[...] += _dot(mask, h4)
    cnt_acc[...] += jnp.sum(mask, axis=1, keepdims=True)

    b0 = bsmem_ref[0, 0, 0]
    b1 = jnp.minimum(bsmem_ref[0, 0, BLK - 1], B - 1)

    def seg_body(b, _):
        nm = bcol == b
        m = jnp.max(jnp.where(nm, h4, -3e38), axis=0, keepdims=True)
        max_acc[pl.ds(b, 1), :] = jnp.maximum(max_acc[pl.ds(b, 1), :], m)
        return 0
    lax.fori_loop(b0, b1 + 1, seg_body, 0)

    @pl.when(i == NBLK - 1)
    def _():
        cnt = cnt_acc[...]
        mean = mean_acc[...] / jnp.maximum(cnt, 1.0)
        mx = jnp.where(cnt > 0.0, max_acc[...], 0.0)
        ge_ref[:, 0:H] = mean
        ge_ref[:, H:2 * H] = mx
        ee = _dot(jnp.maximum(_dot(ef_ref[...], we1_ref[...])
                              + bexp1_ref[...], 0.0),
                  we2_ref[...]) + bexp2_ref[...]
        ee_ref[...] = ee
        cat = jnp.concatenate([mean, mx, ee], axis=1)
        fc = jnp.maximum(_dot(cat, wf1_ref[...]) + bf1_ref[...], 0.0)
        out_ref[...] = _dot(fc, wf2_ref[...]) + bf2_ref[...]


def _tc1(x_pad, deg_col, W1):
    return pl.pallas_call(
        _tc1_body,
        grid=(NBLK,),
        in_specs=[
            pl.BlockSpec((BLK, D_IN), lambda i: (i, 0)),
            pl.BlockSpec((BLK, 1), lambda i: (i, 0)),
            pl.BlockSpec((D_IN, H), lambda i: (0, 0)),
        ],
        out_specs=pl.BlockSpec((4, BLK, 128), lambda i: (0, i, 0)),
        out_shape=jax.ShapeDtypeStruct((4, N_PAD, 128), jnp.float32),
    )(x_pad, deg_col, W1)


def _tcmid(s, deg_col, al, be, W):
    return pl.pallas_call(
        _tcmid_body,
        grid=(NBLK,),
        in_specs=[
            pl.BlockSpec((4, BLK, 128), lambda i: (0, i, 0)),
            pl.BlockSpec((BLK, 1), lambda i: (i, 0)),
            pl.BlockSpec((1, H), lambda i: (0, 0)),
            pl.BlockSpec((1, H), lambda i: (0, 0)),
            pl.BlockSpec((H, H), lambda i: (0, 0)),
        ],
        out_specs=pl.BlockSpec((4, BLK, 128), lambda i: (0, i, 0)),
        out_shape=jax.ShapeDtypeStruct((4, N_PAD, 128), jnp.float32),
    )(s, deg_col, al, be, W)


def _tcfinal(s, deg_col, al, be, brow, bcol, bsmem, ef,
             We1, bexp1, We2, bexp2, Wf1, bf1, Wf2, bf2):
    return pl.pallas_call(
        _tcfinal_body,
        grid=(NBLK,),
        in_specs=[
            pl.BlockSpec((4, BLK, 128), lambda i: (0, i, 0)),
            pl.BlockSpec((BLK, 1), lambda i: (i, 0)),
            pl.BlockSpec((1, H), lambda i: (0, 0)),
            pl.BlockSpec((1, H), lambda i: (0, 0)),
            pl.BlockSpec((1, 1, BLK), lambda i: (i, 0, 0)),
            pl.BlockSpec((BLK, 1), lambda i: (i, 0)),
            pl.BlockSpec((1, 1, BLK), lambda i: (i, 0, 0),
                         memory_space=pltpu.SMEM),
            pl.BlockSpec((B, EXP_D), lambda i: (0, 0)),
            pl.BlockSpec((EXP_D, H), lambda i: (0, 0)),
            pl.BlockSpec((1, H), lambda i: (0, 0)),
            pl.BlockSpec((H, H), lambda i: (0, 0)),
            pl.BlockSpec((1, H), lambda i: (0, 0)),
            pl.BlockSpec((3 * H, 256), lambda i: (0, 0)),
            pl.BlockSpec((1, 256), lambda i: (0, 0)),
            pl.BlockSpec((256, 1), lambda i: (0, 0)),
            pl.BlockSpec((1, 1), lambda i: (0, 0)),
        ],
        out_specs=[
            pl.BlockSpec((B, 1), lambda i: (0, 0)),
            pl.BlockSpec((B, 2 * H), lambda i: (0, 0)),
            pl.BlockSpec((B, H), lambda i: (0, 0)),
        ],
        out_shape=[
            jax.ShapeDtypeStruct((B, 1), jnp.float32),
            jax.ShapeDtypeStruct((B, 2 * H), jnp.float32),
            jax.ShapeDtypeStruct((B, H), jnp.float32),
        ],
        scratch_shapes=[
            pltpu.VMEM((B, H), jnp.float32),
            pltpu.VMEM((B, H), jnp.float32),
            pltpu.VMEM((B, 1), jnp.float32),
        ],
    )(s, deg_col, al, be, brow, bcol, bsmem, ef,
      We1, bexp1, We2, bexp2, Wf1, bf1, Wf2, bf2)


def kernel(x, edge_index, batch, experimental_feat,
           W1, b1, g1, be1, W2, b2, g2, be2, W3, b3, g3, be3,
           W4, b4, g4, be4, We1, bexp1, We2, bexp2, Wf1, bf1, Wf2, bf2):
    x_pad = jnp.pad(x, ((0, N_PAD - N), (0, 0)))
    src = jnp.pad(edge_index[0], (0, E_PAD - E), constant_values=0)
    dst = jnp.pad(edge_index[1], (0, E_PAD - E), constant_values=DUMMY)
    src3 = src.reshape(NS, NBAT, KB)
    dst3 = dst.reshape(NS, NBAT, KB)
    batch_pad = jnp.pad(batch, (0, N_PAD - N), constant_values=B)
    brow = batch_pad.reshape(NBLK, 1, BLK)
    bcol = batch_pad.reshape(N_PAD, 1)

    def fold(g, bconv, be):
        al = (g * _BN_C).reshape(1, H)
        return al, (al * bconv.reshape(1, H) + be.reshape(1, H))

    al1, bt1 = fold(g1, b1, be1)
    al2, bt2 = fold(g2, b2, be2)
    al3, bt3 = fold(g3, b3, be3)
    al4, bt4 = fold(g4, b4, be4)

    deg_parts = _deg_kernel(dst[:E_PAD_DEG])
    deg_col = (deg_parts[0, :, 0] + deg_parts[1, :, 0] + 1.0).reshape(N_PAD, 1)

    h = _tc1(x_pad, deg_col, W1)
    s = _agg_kernel(h, src3, dst3)
    h = _tcmid(s, deg_col, al1, bt1, W2)
    s = _agg_kernel(h, src3, dst3)
    h = _tcmid(s, deg_col, al2, bt2, W3)
    s = _agg_kernel(h, src3, dst3)
    h = _tcmid(s, deg_col, al3, bt3, W4)
    s = _agg_kernel(h, src3, dst3)

    out, graph_emb, exp_emb = _tcfinal(
        s, deg_col, al4, bt4, brow, bcol, brow, experimental_feat,
        We1, bexp1.reshape(1, H), We2, bexp2.reshape(1, H),
        Wf1, bf1.reshape(1, 256), Wf2, bf2.reshape(1, 1))
    return (out, graph_emb, exp_emb)


# speedup vs baseline: 1.2023x; 1.2023x over previous
import functools

import jax
import jax.numpy as jnp
from jax import lax
from jax.experimental import pallas as pl
from jax.experimental.pallas import tpu as pltpu
from jax.experimental.pallas import tpu_sc as plsc

N = 10000
E = 160000
D_IN = 256
H = 512
B = 64
EXP_D = 128

N_PAD = 10240
E_PAD = 163840
NC = 2
NS = 16
KB = 128
ROWS_PER_TILE = N_PAD // NS
EDGES_PER_TILE = E_PAD // NS
NBLK = 10
BLK = N_PAD // NBLK

_BN_C = float(1.0 / (1.0 + 1e-5) ** 0.5)


def _deg_body(dst_hbm, const_hbm, out_hbm, idx_v, ones_v, stage_v, acc_sh,
              sem):
    cid = lax.axis_index("c")
    sid = lax.axis_index("s")

    pltpu.sync_copy(const_hbm.at[0], ones_v)
    pltpu.sync_copy(const_hbm.at[1], stage_v)

    for j in range(ROWS_PER_TILE // KB):
        pltpu.sync_copy(
            stage_v, acc_sh.at[pl.ds(sid * ROWS_PER_TILE + j * KB, KB)])
    plsc.subcore_barrier()

    half = E_PAD // NC
    base = cid * half + sid * (half // NS)
    nbatch = (half // NS) // KB

    def body(j, _):
        pltpu.sync_copy(dst_hbm.at[pl.ds(base + j * KB, KB)], idx_v)
        pltpu.sync_copy(ones_v, acc_sh.at[idx_v], add=True)
        return 0
    lax.fori_loop(0, nbatch, body, 0)
    plsc.subcore_barrier()

    for cc in range(NC):
        @pl.when(cid == cc)
        def _():
            for j in range(ROWS_PER_TILE // KB):
                r0 = sid * ROWS_PER_TILE + j * KB
                pltpu.sync_copy(acc_sh.at[pl.ds(r0, KB)], stage_v)
                pltpu.sync_copy(stage_v, out_hbm.at[cc].at[pl.ds(r0, KB)])


_DEG_W = 128


_EPT_B = EDGES_PER_TILE // KB


def _agg_body(h_hbm, src_hbm, dst_hbm, out_hbm, src_all, dst_ring, rows_v,
              acc_sh, gsem0, gsem1, dsem0, dsem1, ssem0, ssem1):
    cid = lax.axis_index("c")
    sid = lax.axis_index("s")

    pltpu.sync_copy(src_hbm.at[sid], src_all)

    gsems = (gsem0, gsem1)
    dsems = (dsem0, dsem1)
    ssems = (ssem0, ssem1)

    for c in range(4):
        @pl.when(cid == (c // 2))
        def _():
            r0 = sid * ROWS_PER_TILE
            for j in range(ROWS_PER_TILE // KB):
                pltpu.sync_copy(h_hbm.at[c].at[pl.ds(r0 + j * KB, KB)],
                                rows_v.at[0])
                pltpu.sync_copy(rows_v.at[0],
                                acc_sh.at[pl.ds(r0 + j * KB, KB)])
        plsc.subcore_barrier()

        @pl.when(cid == (c // 2))
        def _():
            pltpu.sync_copy(dst_hbm.at[sid].at[0], dst_ring.at[0])
            pltpu.async_copy(h_hbm.at[c].at[src_all.at[0]], rows_v.at[0],
                             gsem0)

            def group(gg, _):
                for b in range(2):
                    j = 2 * gg + b

                    @pl.when(j >= 1)
                    def _():
                        pltpu.make_async_copy(h_hbm.at[c].at[pl.ds(0, KB)],
                                              rows_v.at[1 - b],
                                              ssems[1 - b]).wait()

                    @pl.when(j < _EPT_B - 1)
                    def _():
                        pltpu.async_copy(h_hbm.at[c].at[src_all.at[j + 1]],
                                         rows_v.at[1 - b], gsems[1 - b])
                        pltpu.async_copy(dst_hbm.at[sid].at[j + 1],
                                         dst_ring.at[1 - b], dsems[1 - b])

                    pltpu.make_async_copy(h_hbm.at[c].at[pl.ds(0, KB)],
                                          rows_v.at[b], gsems[b]).wait()

                    @pl.when(j > 0)
                    def _():
                        pltpu.make_async_copy(dst_hbm.at[sid].at[0],
                                              dst_ring.at[b],
                                              dsems[b]).wait()
                    pltpu.async_copy(rows_v.at[b], acc_sh.at[dst_ring.at[b]],
                                     ssems[b], add=True)
                return 0
            lax.fori_loop(0, _EPT_B // 2, group, 0)
            pltpu.make_async_copy(h_hbm.at[c].at[pl.ds(0, KB)],
                                  rows_v.at[1], ssems[1]).wait()
        plsc.subcore_barrier()

        @pl.when(cid == (c // 2))
        def _():
            r0 = sid * ROWS_PER_TILE
            for j in range(ROWS_PER_TILE // KB):
                pltpu.sync_copy(acc_sh.at[pl.ds(r0 + j * KB, KB)],
                                rows_v.at[0])
                pltpu.sync_copy(rows_v.at[0],
                                out_hbm.at[c].at[pl.ds(r0 + j * KB, KB)])
        plsc.subcore_barrier()


@functools.lru_cache(maxsize=None)
def _sc_kernels():
    mesh = plsc.VectorSubcoreMesh(core_axis_name="c", subcore_axis_name="s")
    deg = functools.partial(
        pl.kernel,
        mesh=mesh,
        out_type=jax.ShapeDtypeStruct((NC, N_PAD, _DEG_W), jnp.float32),
        scratch_types=[
            pltpu.VMEM((KB,), jnp.int32),
            pltpu.VMEM((KB, _DEG_W), jnp.float32),
            pltpu.VMEM((KB, _DEG_W), jnp.float32),
            pltpu.VMEM_SHARED((N_PAD, _DEG_W), jnp.float32),
            pltpu.SemaphoreType.DMA,
        ],
    )(_deg_body)
    agg = functools.partial(
        pl.kernel,
        mesh=mesh,
        out_type=jax.ShapeDtypeStruct((4, N_PAD, 128), jnp.float32),
        scratch_types=[
            pltpu.VMEM((_EPT_B, KB), jnp.int32),
            pltpu.VMEM((2, KB), jnp.int32),
            pltpu.VMEM((2, KB, 128), jnp.float32),
            pltpu.VMEM_SHARED((N_PAD, 128), jnp.float32),
        ] + [pltpu.SemaphoreType.DMA] * 6,
    )(_agg_body)
    return deg, agg


def _deg_kernel(dst):
    const = jnp.stack([jnp.ones((KB, _DEG_W), jnp.float32),
                       jnp.zeros((KB, _DEG_W), jnp.float32)])
    return _sc_kernels()[0](dst, const)


def _agg_kernel(h, src3, dst3):
    return _sc_kernels()[1](h, src3, dst3)


def _dinv(deg):
    return jnp.where(deg > 0.0, lax.rsqrt(deg), 0.0)


def _dot(a, b):
    return jnp.dot(a, b, preferred_element_type=jnp.float32,
                   precision=lax.Precision.HIGHEST)


def _tc1_body(x_ref, deg_ref, w_ref, out_ref):
    di = _dinv(deg_ref[...])
    h = _dot(x_ref[...], w_ref[...]) * di
    for c in range(4):
        out_ref[c] = h[:, c * 128:(c + 1) * 128]


def _tcmid_body(s_ref, deg_ref, al_ref, be_ref, w_ref, out_ref):
    di = _dinv(deg_ref[...])
    xb = jnp.concatenate([s_ref[c] for c in range(4)], axis=1)
    a = jnp.maximum(al_ref[...] * (xb * di) + be_ref[...], 0.0)
    h = _dot(a, w_ref[...]) * di
    for c in range(4):
        out_ref[c] = h[:, c * 128:(c + 1) * 128]


def _tcfinal_body(s_ref, deg_ref, al_ref, be_ref, brow_ref, bcol_ref,
                  bsmem_ref, ef_ref, we1_ref, bexp1_ref, we2_ref, bexp2_ref,
                  wf1_ref, bf1_ref, wf2_ref, bf2_ref,
                  out_ref, ge_ref, ee_ref,
                  mean_acc, max_acc, cnt_acc):
    i = pl.program_id(0)

    @pl.when(i == 0)
    def _():
        mean_acc[...] = jnp.zeros((B, H), jnp.float32)
        max_acc[...] = jnp.full((B, H), -3e38, jnp.float32)
        cnt_acc[...] = jnp.zeros((B, 1), jnp.float32)

    di = _dinv(deg_ref[...])
    xb = jnp.concatenate([s_ref[c] for c in range(4)], axis=1)
    h4 = jnp.maximum(al_ref[...] * (xb * di) + be_ref[...], 0.0)

    brow = brow_ref[0]
    seg = lax.broadcasted_iota(jnp.int32, (B, BLK), 0)
    mask = (brow == seg).astype(jnp.float32)
    mean_acc[...] += _dot(mask, h4)
    cnt_acc[...] += jnp.sum(mask, axis=1, keepdims=True)

    bcol = bcol_ref[...]
    b0 = bsmem_ref[0, 0, 0]
    b1 = jnp.minimum(bsmem_ref[0, 0, BLK - 1], B - 1)

    def seg_body(b, _):
        nm = bcol == b
        m = jnp.max(jnp.where(nm, h4, -3e38), axis=0, keepdims=True)
        max_acc[pl.ds(b, 1), :] = jnp.maximum(max_acc[pl.ds(b, 1), :], m)
        return 0
    lax.fori_loop(b0, b1 + 1, seg_body, 0)

    @pl.when(i == NBLK - 1)
    def _():
        cnt = cnt_acc[...]
        mean = mean_acc[...] / jnp.maximum(cnt, 1.0)
        mx = jnp.where(cnt > 0.0, max_acc[...], 0.0)
        ge_ref[:, 0:H] = mean
        ge_ref[:, H:2 * H] = mx
        ee = _dot(jnp.maximum(_dot(ef_ref[...], we1_ref[...])
                              + bexp1_ref[...], 0.0),
                  we2_ref[...]) + bexp2_ref[...]
        ee_ref[...] = ee
        cat = jnp.concatenate([mean, mx, ee], axis=1)
        fc = jnp.maximum(_dot(cat, wf1_ref[...]) + bf1_ref[...], 0.0)
        out_ref[...] = _dot(fc, wf2_ref[...]) + bf2_ref[...]


def _tc1(x_pad, deg_col, W1):
    return pl.pallas_call(
        _tc1_body,
        grid=(NBLK,),
        in_specs=[
            pl.BlockSpec((BLK, D_IN), lambda i: (i, 0)),
            pl.BlockSpec((BLK, 1), lambda i: (i, 0)),
            pl.BlockSpec((D_IN, H), lambda i: (0, 0)),
        ],
        out_specs=pl.BlockSpec((4, BLK, 128), lambda i: (0, i, 0)),
        out_shape=jax.ShapeDtypeStruct((4, N_PAD, 128), jnp.float32),
    )(x_pad, deg_col, W1)


def _tcmid(s, deg_col, al, be, W):
    return pl.pallas_call(
        _tcmid_body,
        grid=(NBLK,),
        in_specs=[
            pl.BlockSpec((4, BLK, 128), lambda i: (0, i, 0)),
            pl.BlockSpec((BLK, 1), lambda i: (i, 0)),
            pl.BlockSpec((1, H), lambda i: (0, 0)),
            pl.BlockSpec((1, H), lambda i: (0, 0)),
            pl.BlockSpec((H, H), lambda i: (0, 0)),
        ],
        out_specs=pl.BlockSpec((4, BLK, 128), lambda i: (0, i, 0)),
        out_shape=jax.ShapeDtypeStruct((4, N_PAD, 128), jnp.float32),
    )(s, deg_col, al, be, W)


def _tcfinal(s, deg_col, al, be, brow, bcol, bsmem, ef,
             We1, bexp1, We2, bexp2, Wf1, bf1, Wf2, bf2):
    return pl.pallas_call(
        _tcfinal_body,
        grid=(NBLK,),
        in_specs=[
            pl.BlockSpec((4, BLK, 128), lambda i: (0, i, 0)),
            pl.BlockSpec((BLK, 1), lambda i: (i, 0)),
            pl.BlockSpec((1, H), lambda i: (0, 0)),
            pl.BlockSpec((1, H), lambda i: (0, 0)),
            pl.BlockSpec((1, 1, BLK), lambda i: (i, 0, 0)),
            pl.BlockSpec((BLK, 1), lambda i: (i, 0)),
            pl.BlockSpec((1, 1, BLK), lambda i: (i, 0, 0),
                         memory_space=pltpu.SMEM),
            pl.BlockSpec((B, EXP_D), lambda i: (0, 0)),
            pl.BlockSpec((EXP_D, H), lambda i: (0, 0)),
            pl.BlockSpec((1, H), lambda i: (0, 0)),
            pl.BlockSpec((H, H), lambda i: (0, 0)),
            pl.BlockSpec((1, H), lambda i: (0, 0)),
            pl.BlockSpec((3 * H, 256), lambda i: (0, 0)),
            pl.BlockSpec((1, 256), lambda i: (0, 0)),
            pl.BlockSpec((256, 1), lambda i: (0, 0)),
            pl.BlockSpec((1, 1), lambda i: (0, 0)),
        ],
        out_specs=[
            pl.BlockSpec((B, 1), lambda i: (0, 0)),
            pl.BlockSpec((B, 2 * H), lambda i: (0, 0)),
            pl.BlockSpec((B, H), lambda i: (0, 0)),
        ],
        out_shape=[
            jax.ShapeDtypeStruct((B, 1), jnp.float32),
            jax.ShapeDtypeStruct((B, 2 * H), jnp.float32),
            jax.ShapeDtypeStruct((B, H), jnp.float32),
        ],
        scratch_shapes=[
            pltpu.VMEM((B, H), jnp.float32),
            pltpu.VMEM((B, H), jnp.float32),
            pltpu.VMEM((B, 1), jnp.float32),
        ],
    )(s, deg_col, al, be, brow, bcol, bsmem, ef,
      We1, bexp1, We2, bexp2, Wf1, bf1, Wf2, bf2)


def kernel(x, edge_index, batch, experimental_feat,
           W1, b1, g1, be1, W2, b2, g2, be2, W3, b3, g3, be3,
           W4, b4, g4, be4, We1, bexp1, We2, bexp2, Wf1, bf1, Wf2, bf2):
    x_pad = jnp.pad(x, ((0, N_PAD - N), (0, 0)))
    src = jnp.pad(edge_index[0], (0, E_PAD - E), constant_values=N_PAD - 1)
    dst = jnp.pad(edge_index[1], (0, E_PAD - E), constant_values=N_PAD - 1)
    src3 = src.reshape(NS, _EPT_B, KB)
    dst3 = dst.reshape(NS, _EPT_B, KB)
    batch_pad = jnp.pad(batch, (0, N_PAD - N), constant_values=B)
    brow = batch_pad.reshape(NBLK, 1, BLK)
    bcol = batch_pad.reshape(N_PAD, 1)

    def fold(g, bconv, be):
        al = (g * _BN_C).reshape(1, H)
        return al, (al * bconv.reshape(1, H) + be.reshape(1, H))

    al1, bt1 = fold(g1, b1, be1)
    al2, bt2 = fold(g2, b2, be2)
    al3, bt3 = fold(g3, b3, be3)
    al4, bt4 = fold(g4, b4, be4)

    deg_parts = _deg_kernel(dst)
    deg_col = (deg_parts[0, :, 0] + deg_parts[1, :, 0] + 1.0).reshape(N_PAD, 1)

    h = _tc1(x_pad, deg_col, W1)
    s = _agg_kernel(h, src3, dst3)
    h = _tcmid(s, deg_col, al1, bt1, W2)
    s = _agg_kernel(h, src3, dst3)
    h = _tcmid(s, deg_col, al2, bt2, W3)
    s = _agg_kernel(h, src3, dst3)
    h = _tcmid(s, deg_col, al3, bt3, W4)
    s = _agg_kernel(h, src3, dst3)

    out, graph_emb, exp_emb = _tcfinal(
        s, deg_col, al4, bt4, brow, bcol, brow, experimental_feat,
        We1, bexp1.reshape(1, H), We2, bexp2.reshape(1, H),
        Wf1, bf1.reshape(1, 256), Wf2, bf2.reshape(1, 1))
    return (out, graph_emb, exp_emb)


# speedup vs baseline: 1.2207x; 1.0153x over previous
import functools

import jax
import jax.numpy as jnp
from jax import lax
from jax.experimental import pallas as pl
from jax.experimental.pallas import tpu as pltpu
from jax.experimental.pallas import tpu_sc as plsc

N = 10000
E = 160000
D_IN = 256
H = 512
B = 64
EXP_D = 128

N_PAD = 10240
E_PAD = 163840
NC = 2
NS = 16
KB = 128
ROWS_PER_TILE = N_PAD // NS
EDGES_PER_TILE = E_PAD // NS
NBLK = 10
BLK = N_PAD // NBLK

_BN_C = float(1.0 / (1.0 + 1e-5) ** 0.5)


def _deg_body(dst_hbm, const_hbm, out_hbm, idx_v, ones_v, stage_v, acc_sh,
              sem):
    cid = lax.axis_index("c")
    sid = lax.axis_index("s")

    pltpu.sync_copy(const_hbm.at[0], ones_v)
    pltpu.sync_copy(const_hbm.at[1], stage_v)

    for j in range(ROWS_PER_TILE // KB):
        pltpu.sync_copy(
            stage_v, acc_sh.at[pl.ds(sid * ROWS_PER_TILE + j * KB, KB)])
    plsc.subcore_barrier()

    half = E_PAD // NC
    base = cid * half + sid * (half // NS)
    nbatch = (half // NS) // KB

    def body(j, _):
        pltpu.sync_copy(dst_hbm.at[pl.ds(base + j * KB, KB)], idx_v)
        pltpu.sync_copy(ones_v, acc_sh.at[idx_v], add=True)
        return 0
    lax.fori_loop(0, nbatch, body, 0)
    plsc.subcore_barrier()

    for cc in range(NC):
        @pl.when(cid == cc)
        def _():
            for j in range(ROWS_PER_TILE // KB):
                r0 = sid * ROWS_PER_TILE + j * KB
                pltpu.sync_copy(acc_sh.at[pl.ds(r0, KB)], stage_v)
                pltpu.sync_copy(stage_v, out_hbm.at[cc].at[pl.ds(r0, KB)])


_DEG_W = 128


_EPT_B = EDGES_PER_TILE // KB


def _agg_body(h_hbm, src_hbm, dst_hbm, out_hbm, src_all, dst_ring, rows_v,
              acc_sh, gsem0, gsem1, dsem0, dsem1, ssem0, ssem1):
    cid = lax.axis_index("c")
    sid = lax.axis_index("s")

    pltpu.sync_copy(src_hbm.at[sid], src_all)

    gsems = (gsem0, gsem1)
    dsems = (dsem0, dsem1)
    ssems = (ssem0, ssem1)

    for c in range(4):
        @pl.when(cid == (c // 2))
        def _():
            r0 = sid * ROWS_PER_TILE
            pltpu.sync_copy(h_hbm.at[c].at[pl.ds(r0, ROWS_PER_TILE)],
                            acc_sh.at[pl.ds(r0, ROWS_PER_TILE)])
        plsc.subcore_barrier()

        @pl.when(cid == (c // 2))
        def _():
            pltpu.sync_copy(dst_hbm.at[sid].at[0], dst_ring.at[0])
            pltpu.async_copy(h_hbm.at[c].at[src_all.at[0]], rows_v.at[0],
                             gsem0)

            def group(gg, _):
                for b in range(2):
                    j = 2 * gg + b

                    @pl.when(j >= 1)
                    def _():
                        pltpu.make_async_copy(h_hbm.at[c].at[pl.ds(0, KB)],
                                              rows_v.at[1 - b],
                                              ssems[1 - b]).wait()

                    @pl.when(j < _EPT_B - 1)
                    def _():
                        pltpu.async_copy(h_hbm.at[c].at[src_all.at[j + 1]],
                                         rows_v.at[1 - b], gsems[1 - b])
                        pltpu.async_copy(dst_hbm.at[sid].at[j + 1],
                                         dst_ring.at[1 - b], dsems[1 - b])

                    pltpu.make_async_copy(h_hbm.at[c].at[pl.ds(0, KB)],
                                          rows_v.at[b], gsems[b]).wait()

                    @pl.when(j > 0)
                    def _():
                        pltpu.make_async_copy(dst_hbm.at[sid].at[0],
                                              dst_ring.at[b],
                                              dsems[b]).wait()
                    pltpu.async_copy(rows_v.at[b], acc_sh.at[dst_ring.at[b]],
                                     ssems[b], add=True)
                return 0
            lax.fori_loop(0, _EPT_B // 2, group, 0)
            pltpu.make_async_copy(h_hbm.at[c].at[pl.ds(0, KB)],
                                  rows_v.at[1], ssems[1]).wait()
        plsc.subcore_barrier()

        @pl.when(cid == (c // 2))
        def _():
            r0 = sid * ROWS_PER_TILE
            pltpu.sync_copy(acc_sh.at[pl.ds(r0, ROWS_PER_TILE)],
                            out_hbm.at[c].at[pl.ds(r0, ROWS_PER_TILE)])
        plsc.subcore_barrier()


@functools.lru_cache(maxsize=None)
def _sc_kernels():
    mesh = plsc.VectorSubcoreMesh(core_axis_name="c", subcore_axis_name="s")
    deg = functools.partial(
        pl.kernel,
        mesh=mesh,
        out_type=jax.ShapeDtypeStruct((NC, N_PAD, _DEG_W), jnp.float32),
        scratch_types=[
            pltpu.VMEM((KB,), jnp.int32),
            pltpu.VMEM((KB, _DEG_W), jnp.float32),
            pltpu.VMEM((KB, _DEG_W), jnp.float32),
            pltpu.VMEM_SHARED((N_PAD, _DEG_W), jnp.float32),
            pltpu.SemaphoreType.DMA,
        ],
    )(_deg_body)
    agg = functools.partial(
        pl.kernel,
        mesh=mesh,
        out_type=jax.ShapeDtypeStruct((4, N_PAD, 128), jnp.float32),
        scratch_types=[
            pltpu.VMEM((_EPT_B, KB), jnp.int32),
            pltpu.VMEM((2, KB), jnp.int32),
            pltpu.VMEM((2, KB, 128), jnp.float32),
            pltpu.VMEM_SHARED((N_PAD, 128), jnp.float32),
        ] + [pltpu.SemaphoreType.DMA] * 6,
    )(_agg_body)
    return deg, agg


def _deg_kernel(dst):
    const = jnp.stack([jnp.ones((KB, _DEG_W), jnp.float32),
                       jnp.zeros((KB, _DEG_W), jnp.float32)])
    return _sc_kernels()[0](dst, const)


def _agg_kernel(h, src3, dst3):
    return _sc_kernels()[1](h, src3, dst3)


def _dinv(deg):
    return jnp.where(deg > 0.0, lax.rsqrt(deg), 0.0)


def _dot(a, b):
    return jnp.dot(a, b, preferred_element_type=jnp.float32,
                   precision=lax.Precision.HIGHEST)


def _tc1_body(x_ref, deg_ref, w_ref, out_ref):
    di = _dinv(deg_ref[...])
    h = _dot(x_ref[...], w_ref[...]) * di
    for c in range(4):
        out_ref[c] = h[:, c * 128:(c + 1) * 128]


def _tcmid_body(s_ref, deg_ref, al_ref, be_ref, w_ref, out_ref):
    di = _dinv(deg_ref[...])
    xb = jnp.concatenate([s_ref[c] for c in range(4)], axis=1)
    a = jnp.maximum(al_ref[...] * (xb * di) + be_ref[...], 0.0)
    h = _dot(a, w_ref[...]) * di
    for c in range(4):
        out_ref[c] = h[:, c * 128:(c + 1) * 128]


def _tcfinal_body(s_ref, deg_ref, al_ref, be_ref, brow_ref, bcol_ref,
                  bsmem_ref, ef_ref, we1_ref, bexp1_ref, we2_ref, bexp2_ref,
                  wf1_ref, bf1_ref, wf2_ref, bf2_ref,
                  out_ref, ge_ref, ee_ref,
                  mean_acc, max_acc, cnt_acc):
    i = pl.program_id(0)

    @pl.when(i == 0)
    def _():
        mean_acc[...] = jnp.zeros((B, H), jnp.float32)
        max_acc[...] = jnp.full((B, H), -3e38, jnp.float32)
        cnt_acc[...] = jnp.zeros((B, 1), jnp.float32)

    di = _dinv(deg_ref[...])
    xb = jnp.concatenate([s_ref[c] for c in range(4)], axis=1)
    h4 = jnp.maximum(al_ref[...] * (xb * di) + be_ref[...], 0.0)

    brow = brow_ref[0]
    seg = lax.broadcasted_iota(jnp.int32, (B, BLK), 0)
    mask = (brow == seg).astype(jnp.float32)
    mean_acc[...] += _dot(mask, h4)
    cnt_acc[...] += jnp.sum(mask, axis=1, keepdims=True)

    bcol = bcol_ref[...]
    b0 = bsmem_ref[0, 0, 0]
    b1 = jnp.minimum(bsmem_ref[0, 0, BLK - 1], B - 1)

    def seg_body(b, _):
        nm = bcol == b
        m = jnp.max(jnp.where(nm, h4, -3e38), axis=0, keepdims=True)
        max_acc[pl.ds(b, 1), :] = jnp.maximum(max_acc[pl.ds(b, 1), :], m)
        return 0
    lax.fori_loop(b0, b1 + 1, seg_body, 0)

    @pl.when(i == NBLK - 1)
    def _():
        cnt = cnt_acc[...]
        mean = mean_acc[...] / jnp.maximum(cnt, 1.0)
        mx = jnp.where(cnt > 0.0, max_acc[...], 0.0)
        ge_ref[:, 0:H] = mean
        ge_ref[:, H:2 * H] = mx
        ee = _dot(jnp.maximum(_dot(ef_ref[...], we1_ref[...])
                              + bexp1_ref[...], 0.0),
                  we2_ref[...]) + bexp2_ref[...]
        ee_ref[...] = ee
        cat = jnp.concatenate([mean, mx, ee], axis=1)
        fc = jnp.maximum(_dot(cat, wf1_ref[...]) + bf1_ref[...], 0.0)
        out_ref[...] = _dot(fc, wf2_ref[...]) + bf2_ref[...]


def _tc1(x_pad, deg_col, W1):
    return pl.pallas_call(
        _tc1_body,
        grid=(NBLK,),
        in_specs=[
            pl.BlockSpec((BLK, D_IN), lambda i: (i, 0)),
            pl.BlockSpec((BLK, 1), lambda i: (i, 0)),
            pl.BlockSpec((D_IN, H), lambda i: (0, 0)),
        ],
        out_specs=pl.BlockSpec((4, BLK, 128), lambda i: (0, i, 0)),
        out_shape=jax.ShapeDtypeStruct((4, N_PAD, 128), jnp.float32),
    )(x_pad, deg_col, W1)


def _tcmid(s, deg_col, al, be, W):
    return pl.pallas_call(
        _tcmid_body,
        grid=(NBLK,),
        in_specs=[
            pl.BlockSpec((4, BLK, 128), lambda i: (0, i, 0)),
            pl.BlockSpec((BLK, 1), lambda i: (i, 0)),
            pl.BlockSpec((1, H), lambda i: (0, 0)),
            pl.BlockSpec((1, H), lambda i: (0, 0)),
            pl.BlockSpec((H, H), lambda i: (0, 0)),
        ],
        out_specs=pl.BlockSpec((4, BLK, 128), lambda i: (0, i, 0)),
        out_shape=jax.ShapeDtypeStruct((4, N_PAD, 128), jnp.float32),
    )(s, deg_col, al, be, W)


def _tcfinal(s, deg_col, al, be, brow, bcol, bsmem, ef,
             We1, bexp1, We2, bexp2, Wf1, bf1, Wf2, bf2):
    return pl.pallas_call(
        _tcfinal_body,
        grid=(NBLK,),
        in_specs=[
            pl.BlockSpec((4, BLK, 128), lambda i: (0, i, 0)),
            pl.BlockSpec((BLK, 1), lambda i: (i, 0)),
            pl.BlockSpec((1, H), lambda i: (0, 0)),
            pl.BlockSpec((1, H), lambda i: (0, 0)),
            pl.BlockSpec((1, 1, BLK), lambda i: (i, 0, 0)),
            pl.BlockSpec((BLK, 1), lambda i: (i, 0)),
            pl.BlockSpec((1, 1, BLK), lambda i: (i, 0, 0),
                         memory_space=pltpu.SMEM),
            pl.BlockSpec((B, EXP_D), lambda i: (0, 0)),
            pl.BlockSpec((EXP_D, H), lambda i: (0, 0)),
            pl.BlockSpec((1, H), lambda i: (0, 0)),
            pl.BlockSpec((H, H), lambda i: (0, 0)),
            pl.BlockSpec((1, H), lambda i: (0, 0)),
            pl.BlockSpec((3 * H, 256), lambda i: (0, 0)),
            pl.BlockSpec((1, 256), lambda i: (0, 0)),
            pl.BlockSpec((256, 1), lambda i: (0, 0)),
            pl.BlockSpec((1, 1), lambda i: (0, 0)),
        ],
        out_specs=[
            pl.BlockSpec((B, 1), lambda i: (0, 0)),
            pl.BlockSpec((B, 2 * H), lambda i: (0, 0)),
            pl.BlockSpec((B, H), lambda i: (0, 0)),
        ],
        out_shape=[
            jax.ShapeDtypeStruct((B, 1), jnp.float32),
            jax.ShapeDtypeStruct((B, 2 * H), jnp.float32),
            jax.ShapeDtypeStruct((B, H), jnp.float32),
        ],
        scratch_shapes=[
            pltpu.VMEM((B, H), jnp.float32),
            pltpu.VMEM((B, H), jnp.float32),
            pltpu.VMEM((B, 1), jnp.float32),
        ],
    )(s, deg_col, al, be, brow, bcol, bsmem, ef,
      We1, bexp1, We2, bexp2, Wf1, bf1, Wf2, bf2)


def kernel(x, edge_index, batch, experimental_feat,
           W1, b1, g1, be1, W2, b2, g2, be2, W3, b3, g3, be3,
           W4, b4, g4, be4, We1, bexp1, We2, bexp2, Wf1, bf1, Wf2, bf2):
    x_pad = jnp.pad(x, ((0, N_PAD - N), (0, 0)))
    src = jnp.pad(edge_index[0], (0, E_PAD - E), constant_values=N_PAD - 1)
    dst = jnp.pad(edge_index[1], (0, E_PAD - E), constant_values=N_PAD - 1)
    src3 = src.reshape(NS, _EPT_B, KB)
    dst3 = dst.reshape(NS, _EPT_B, KB)
    batch_pad = jnp.pad(batch, (0, N_PAD - N), constant_values=B)
    brow = batch_pad.reshape(NBLK, 1, BLK)
    bcol = batch_pad.reshape(N_PAD, 1)

    def fold(g, bconv, be):
        al = (g * _BN_C).reshape(1, H)
        return al, (al * bconv.reshape(1, H) + be.reshape(1, H))

    al1, bt1 = fold(g1, b1, be1)
    al2, bt2 = fold(g2, b2, be2)
    al3, bt3 = fold(g3, b3, be3)
    al4, bt4 = fold(g4, b4, be4)

    deg_parts = _deg_kernel(dst)
    deg_col = (deg_parts[0, :, 0] + deg_parts[1, :, 0] + 1.0).reshape(N_PAD, 1)

    h = _tc1(x_pad, deg_col, W1)
    s = _agg_kernel(h, src3, dst3)
    h = _tcmid(s, deg_col, al1, bt1, W2)
    s = _agg_kernel(h, src3, dst3)
    h = _tcmid(s, deg_col, al2, bt2, W3)
    s = _agg_kernel(h, src3, dst3)
    h = _tcmid(s, deg_col, al3, bt3, W4)
    s = _agg_kernel(h, src3, dst3)

    out, graph_emb, exp_emb = _tcfinal(
        s, deg_col, al4, bt4, brow, bcol, brow, experimental_feat,
        We1, bexp1.reshape(1, H), We2, bexp2.reshape(1, H),
        Wf1, bf1.reshape(1, 256), Wf2, bf2.reshape(1, 1))
    return (out, graph_emb, exp_emb)


# speedup vs baseline: 1.4289x; 1.1705x over previous
import functools

import jax
import jax.numpy as jnp
from jax import lax
from jax.experimental import pallas as pl
from jax.experimental.pallas import tpu as pltpu
from jax.experimental.pallas import tpu_sc as plsc

N = 10000
E = 160000
D_IN = 256
H = 512
B = 64
EXP_D = 128

N_PAD = 10240
E_PAD = 162816
E_PAD_DEG = 163840
NC = 2
NS = 16
KB = 96
NBAT = E_PAD // NS // KB
ROWS_PER_TILE = N_PAD // NS
ACC_ROWS = 10112
ACC_RPT = ACC_ROWS // NS
DUMMY = 10100
NBLK = 10
BLK = N_PAD // NBLK

_BN_C = float(1.0 / (1.0 + 1e-5) ** 0.5)


def _deg_body(dst_hbm, const_hbm, out_hbm, idx_v, ones_v, stage_v, acc_sh,
              sem):
    cid = lax.axis_index("c")
    sid = lax.axis_index("s")

    pltpu.sync_copy(const_hbm.at[0], ones_v)
    pltpu.sync_copy(const_hbm.at[1], stage_v)

    for j in range(ROWS_PER_TILE // 128):
        pltpu.sync_copy(
            stage_v, acc_sh.at[pl.ds(sid * ROWS_PER_TILE + j * 128, 128)])
    plsc.subcore_barrier()

    half = E_PAD_DEG // NC
    base = cid * half + sid * (half // NS)
    nbatch = (half // NS) // 128

    def body(j, _):
        pltpu.sync_copy(dst_hbm.at[pl.ds(base + j * 128, 128)], idx_v)
        pltpu.sync_copy(ones_v, acc_sh.at[idx_v], add=True)
        return 0
    lax.fori_loop(0, nbatch, body, 0)
    plsc.subcore_barrier()

    for cc in range(NC):
        @pl.when(cid == cc)
        def _():
            for j in range(ROWS_PER_TILE // 128):
                r0 = sid * ROWS_PER_TILE + j * 128
                pltpu.sync_copy(acc_sh.at[pl.ds(r0, 128)], stage_v)
                pltpu.sync_copy(stage_v, out_hbm.at[cc].at[pl.ds(r0, 128)])


_DEG_W = 128


def _agg_body(h_hbm, src_hbm, dst_hbm, out_hbm, src_all, dst_all, rows_v,
              acc_sh, gsem0, gsem1, ssem0, ssem1):
    cid = lax.axis_index("c")
    sid = lax.axis_index("s")

    pltpu.sync_copy(src_hbm.at[sid], src_all)
    pltpu.sync_copy(dst_hbm.at[sid], dst_all)

    gsems = (gsem0, gsem1)
    ssems = (ssem0, ssem1)

    for c in range(4):
        @pl.when(cid == (c // 2))
        def _():
            r0 = sid * ACC_RPT
            pltpu.sync_copy(h_hbm.at[c].at[pl.ds(r0, ACC_RPT)],
                            acc_sh.at[pl.ds(r0, ACC_RPT)])
        plsc.subcore_barrier()

        @pl.when(cid == (c // 2))
        def _():
            pltpu.async_copy(h_hbm.at[c].at[src_all.at[pl.ds(0, KB)]],
                             rows_v.at[0], gsem0)

            def group(gg, _):
                for b in range(2):
                    j = 2 * gg + b

                    @pl.when(j >= 1)
                    def _():
                        pltpu.make_async_copy(h_hbm.at[c].at[pl.ds(0, KB)],
                                              rows_v.at[1 - b],
                                              ssems[1 - b]).wait()

                    @pl.when(j < NBAT - 1)
                    def _():
                        pltpu.async_copy(
                            h_hbm.at[c].at[src_all.at[pl.ds((j + 1) * KB,
                                                            KB)]],
                            rows_v.at[1 - b], gsems[1 - b])

                    pltpu.make_async_copy(h_hbm.at[c].at[pl.ds(0, KB)],
                                          rows_v.at[b], gsems[b]).wait()
                    pltpu.async_copy(rows_v.at[b], acc_sh.at[dst_all.at[j]],
                                     ssems[b], add=True)
                return 0
            lax.fori_loop(0, NBAT // 2, group, 0)
            pltpu.make_async_copy(h_hbm.at[c].at[pl.ds(0, KB)],
                                  rows_v.at[(NBAT - 1) % 2],
                                  ssems[(NBAT - 1) % 2]).wait()
        plsc.subcore_barrier()

        @pl.when(cid == (c // 2))
        def _():
            r0 = sid * ACC_RPT
            pltpu.sync_copy(acc_sh.at[pl.ds(r0, ACC_RPT)],
                            out_hbm.at[c].at[pl.ds(r0, ACC_RPT)])
        plsc.subcore_barrier()


@functools.lru_cache(maxsize=None)
def _sc_kernels():
    mesh = plsc.VectorSubcoreMesh(core_axis_name="c", subcore_axis_name="s")
    deg = functools.partial(
        pl.kernel,
        mesh=mesh,
        out_type=jax.ShapeDtypeStruct((NC, N_PAD, _DEG_W), jnp.float32),
        scratch_types=[
            pltpu.VMEM((128,), jnp.int32),
            pltpu.VMEM((128, _DEG_W), jnp.float32),
            pltpu.VMEM((128, _DEG_W), jnp.float32),
            pltpu.VMEM_SHARED((N_PAD, _DEG_W), jnp.float32),
            pltpu.SemaphoreType.DMA,
        ],
    )(_deg_body)
    agg = functools.partial(
        pl.kernel,
        mesh=mesh,
        out_type=jax.ShapeDtypeStruct((4, N_PAD, 128), jnp.float32),
        scratch_types=[
            pltpu.VMEM((NBAT * KB,), jnp.int32),
            pltpu.VMEM((NBAT, KB), jnp.int32),
            pltpu.VMEM((2, KB, 128), jnp.float32),
            pltpu.VMEM_SHARED((ACC_ROWS, 128), jnp.float32),
        ] + [pltpu.SemaphoreType.DMA] * 4,
    )(_agg_body)
    return deg, agg


def _deg_kernel(dst):
    const = jnp.stack([jnp.ones((128, _DEG_W), jnp.float32),
                       jnp.zeros((128, _DEG_W), jnp.float32)])
    return _sc_kernels()[0](dst, const)


def _agg_kernel(h, src3, dst3):
    return _sc_kernels()[1](h, src3, dst3)


def _dinv(deg):
    return jnp.where(deg > 0.0, lax.rsqrt(deg), 0.0)


def _dot(a, b):
    return jnp.dot(a, b, preferred_element_type=jnp.float32,
                   precision=lax.Precision.HIGHEST)


def _tc1_body(x_ref, deg_ref, w_ref, out_ref):
    di = _dinv(deg_ref[...])
    h = _dot(x_ref[...], w_ref[...]) * di
    for c in range(4):
        out_ref[c] = h[:, c * 128:(c + 1) * 128]


def _tcmid_body(s_ref, deg_ref, al_ref, be_ref, w_ref, out_ref):
    di = _dinv(deg_ref[...])
    xb = jnp.concatenate([s_ref[c] for c in range(4)], axis=1)
    a = jnp.maximum(al_ref[...] * (xb * di) + be_ref[...], 0.0)
    h = _dot(a, w_ref[...]) * di
    for c in range(4):
        out_ref[c] = h[:, c * 128:(c + 1) * 128]


def _tcfinal_body(s_ref, deg_ref, al_ref, be_ref, brow_ref, bcol_ref,
                  bsmem_ref, ef_ref, we1_ref, bexp1_ref, we2_ref, bexp2_ref,
                  wf1_ref, bf1_ref, wf2_ref, bf2_ref,
                  out_ref, ge_ref, ee_ref,
                  mean_acc, max_acc, cnt_acc):
    i = pl.program_id(0)

    @pl.when(i == 0)
    def _():
        mean_acc[...] = jnp.zeros((B, H), jnp.float32)
        max_acc[...] = jnp.full((B, H), -3e38, jnp.float32)
        cnt_acc[...] = jnp.zeros((B, 1), jnp.float32)

    di = _dinv(deg_ref[...])
    xb = jnp.concatenate([s_ref[c] for c in range(4)], axis=1)
    h4 = jnp.maximum(al_ref[...] * (xb * di) + be_ref[...], 0.0)
    h4 = jnp.where(bcol_ref[...] < B, h4, 0.0)

    brow = brow_ref[0]
    seg = lax.broadcasted_iota(jnp.int32, (B, BLK), 0)
    mask = (brow == seg).astype(jnp.float32)
    mean_acc[...] += _dot(mask, h4)
    cnt_acc[...] += jnp.sum(mask, axis=1, keepdims=True)

    bcol = bcol_ref[...]
    b0 = bsmem_ref[0, 0, 0]
    b1 = jnp.minimum(bsmem_ref[0, 0, BLK - 1], B - 1)

    def seg_body(b, _):
        nm = bcol == b
        m = jnp.max(jnp.where(nm, h4, -3e38), axis=0, keepdims=True)
        max_acc[pl.ds(b, 1), :] = jnp.maximum(max_acc[pl.ds(b, 1), :], m)
        return 0
    lax.fori_loop(b0, b1 + 1, seg_body, 0)

    @pl.when(i == NBLK - 1)
    def _():
        cnt = cnt_acc[...]
        mean = mean_acc[...] / jnp.maximum(cnt, 1.0)
        mx = jnp.where(cnt > 0.0, max_acc[...], 0.0)
        ge_ref[:, 0:H] = mean
        ge_ref[:, H:2 * H] = mx
        ee = _dot(jnp.maximum(_dot(ef_ref[...], we1_ref[...])
                              + bexp1_ref[...], 0.0),
                  we2_ref[...]) + bexp2_ref[...]
        ee_ref[...] = ee
        cat = jnp.concatenate([mean, mx, ee], axis=1)
        fc = jnp.maximum(_dot(cat, wf1_ref[...]) + bf1_ref[...], 0.0)
        out_ref[...] = _dot(fc, wf2_ref[...]) + bf2_ref[...]


def _tc1(x_pad, deg_col, W1):
    return pl.pallas_call(
        _tc1_body,
        grid=(NBLK,),
        in_specs=[
            pl.BlockSpec((BLK, D_IN), lambda i: (i, 0)),
            pl.BlockSpec((BLK, 1), lambda i: (i, 0)),
            pl.BlockSpec((D_IN, H), lambda i: (0, 0)),
        ],
        out_specs=pl.BlockSpec((4, BLK, 128), lambda i: (0, i, 0)),
        out_shape=jax.ShapeDtypeStruct((4, N_PAD, 128), jnp.float32),
    )(x_pad, deg_col, W1)


def _tcmid(s, deg_col, al, be, W):
    return pl.pallas_call(
        _tcmid_body,
        grid=(NBLK,),
        in_specs=[
            pl.BlockSpec((4, BLK, 128), lambda i: (0, i, 0)),
            pl.BlockSpec((BLK, 1), lambda i: (i, 0)),
            pl.BlockSpec((1, H), lambda i: (0, 0)),
            pl.BlockSpec((1, H), lambda i: (0, 0)),
            pl.BlockSpec((H, H), lambda i: (0, 0)),
        ],
        out_specs=pl.BlockSpec((4, BLK, 128), lambda i: (0, i, 0)),
        out_shape=jax.ShapeDtypeStruct((4, N_PAD, 128), jnp.float32),
    )(s, deg_col, al, be, W)


def _tcfinal(s, deg_col, al, be, brow, bcol, bsmem, ef,
             We1, bexp1, We2, bexp2, Wf1, bf1, Wf2, bf2):
    return pl.pallas_call(
        _tcfinal_body,
        grid=(NBLK,),
        in_specs=[
            pl.BlockSpec((4, BLK, 128), lambda i: (0, i, 0)),
            pl.BlockSpec((BLK, 1), lambda i: (i, 0)),
            pl.BlockSpec((1, H), lambda i: (0, 0)),
            pl.BlockSpec((1, H), lambda i: (0, 0)),
            pl.BlockSpec((1, 1, BLK), lambda i: (i, 0, 0)),
            pl.BlockSpec((BLK, 1), lambda i: (i, 0)),
            pl.BlockSpec((1, 1, BLK), lambda i: (i, 0, 0),
                         memory_space=pltpu.SMEM),
            pl.BlockSpec((B, EXP_D), lambda i: (0, 0)),
            pl.BlockSpec((EXP_D, H), lambda i: (0, 0)),
            pl.BlockSpec((1, H), lambda i: (0, 0)),
            pl.BlockSpec((H, H), lambda i: (0, 0)),
            pl.BlockSpec((1, H), lambda i: (0, 0)),
            pl.BlockSpec((3 * H, 256), lambda i: (0, 0)),
            pl.BlockSpec((1, 256), lambda i: (0, 0)),
            pl.BlockSpec((256, 1), lambda i: (0, 0)),
            pl.BlockSpec((1, 1), lambda i: (0, 0)),
        ],
        out_specs=[
            pl.BlockSpec((B, 1), lambda i: (0, 0)),
            pl.BlockSpec((B, 2 * H), lambda i: (0, 0)),
            pl.BlockSpec((B, H), lambda i: (0, 0)),
        ],
        out_shape=[
            jax.ShapeDtypeStruct((B, 1), jnp.float32),
            jax.ShapeDtypeStruct((B, 2 * H), jnp.float32),
            jax.ShapeDtypeStruct((B, H), jnp.float32),
        ],
        scratch_shapes=[
            pltpu.VMEM((B, H), jnp.float32),
            pltpu.VMEM((B, H), jnp.float32),
            pltpu.VMEM((B, 1), jnp.float32),
        ],
    )(s, deg_col, al, be, brow, bcol, bsmem, ef,
      We1, bexp1, We2, bexp2, Wf1, bf1, Wf2, bf2)


def kernel(x, edge_index, batch, experimental_feat,
           W1, b1, g1, be1, W2, b2, g2, be2, W3, b3, g3, be3,
           W4, b4, g4, be4, We1, bexp1, We2, bexp2, Wf1, bf1, Wf2, bf2):
    x_pad = jnp.pad(x, ((0, N_PAD - N), (0, 0)))
    src3 = jnp.pad(edge_index[0], (0, E_PAD - E),
                   constant_values=0).reshape(NS, NBAT * KB)
    dst_pad = jnp.pad(edge_index[1], (0, E_PAD - E), constant_values=DUMMY)
    dst3 = dst_pad.reshape(NS, NBAT, KB)
    dst_deg = jnp.pad(edge_index[1], (0, E_PAD_DEG - E),
                      constant_values=DUMMY)
    batch_pad = jnp.pad(batch, (0, N_PAD - N), constant_values=B)
    brow = batch_pad.reshape(NBLK, 1, BLK)
    bcol = batch_pad.reshape(N_PAD, 1)

    def fold(g, bconv, be):
        al = (g * _BN_C).reshape(1, H)
        return al, (al * bconv.reshape(1, H) + be.reshape(1, H))

    al1, bt1 = fold(g1, b1, be1)
    al2, bt2 = fold(g2, b2, be2)
    al3, bt3 = fold(g3, b3, be3)
    al4, bt4 = fold(g4, b4, be4)

    deg_parts = _deg_kernel(dst_deg)
    deg_col = (deg_parts[0, :, 0] + deg_parts[1, :, 0] + 1.0).reshape(N_PAD, 1)

    h = _tc1(x_pad, deg_col, W1)
    s = _agg_kernel(h, src3, dst3)
    h = _tcmid(s, deg_col, al1, bt1, W2)
    s = _agg_kernel(h, src3, dst3)
    h = _tcmid(s, deg_col, al2, bt2, W3)
    s = _agg_kernel(h, src3, dst3)
    h = _tcmid(s, deg_col, al3, bt3, W4)
    s = _agg_kernel(h, src3, dst3)

    out, graph_emb, exp_emb = _tcfinal(
        s, deg_col, al4, bt4, brow, bcol, brow, experimental_feat,
        We1, bexp1.reshape(1, H), We2, bexp2.reshape(1, H),
        Wf1, bf1.reshape(1, 256), Wf2, bf2.reshape(1, 1))
    return (out, graph_emb, exp_emb)


# speedup vs baseline: 1.4394x; 1.0074x over previous
import functools

import jax
import jax.numpy as jnp
from jax import lax
from jax.experimental import pallas as pl
from jax.experimental.pallas import tpu as pltpu
from jax.experimental.pallas import tpu_sc as plsc

N = 10000
E = 160000
D_IN = 256
H = 512
B = 64
EXP_D = 128

N_PAD = 10240
E_PAD = 162816
E_PAD_DEG = 163840
NC = 2
NS = 16
KB = 96
NBAT = E_PAD // NS // KB
ROWS_PER_TILE = N_PAD // NS
ACC_ROWS = 10112
ACC_RPT = ACC_ROWS // NS
DUMMY = 10100
NBLK = 10
BLK = N_PAD // NBLK

_BN_C = float(1.0 / (1.0 + 1e-5) ** 0.5)


_DEG_NB = E_PAD_DEG // NC // NS // 128


def _deg_body(dst_hbm, const_hbm, out_hbm, idx_all, ones_v, stage_v, acc_sh,
              sem):
    cid = lax.axis_index("c")
    sid = lax.axis_index("s")

    pltpu.sync_copy(const_hbm.at[0], ones_v)
    pltpu.sync_copy(const_hbm.at[1], stage_v)
    pltpu.sync_copy(dst_hbm.at[cid].at[sid], idx_all)

    for j in range(ROWS_PER_TILE // 128):
        pltpu.sync_copy(
            stage_v, acc_sh.at[pl.ds(sid * ROWS_PER_TILE + j * 128, 128)])
    plsc.subcore_barrier()

    def body(j, _):
        pltpu.async_copy(ones_v, acc_sh.at[idx_all.at[j]], sem, add=True)
        return 0
    lax.fori_loop(0, _DEG_NB, body, 0)

    def dbody(j, _):
        pltpu.make_async_copy(const_hbm.at[0], ones_v, sem).wait()
        return 0
    lax.fori_loop(0, _DEG_NB, dbody, 0)
    plsc.subcore_barrier()

    r0 = sid * ROWS_PER_TILE
    pltpu.sync_copy(acc_sh.at[pl.ds(r0, ROWS_PER_TILE)],
                    out_hbm.at[cid].at[pl.ds(r0, ROWS_PER_TILE)])


_DEG_W = 128


def _agg_body(h_hbm, src_hbm, dst_hbm, out_hbm, src_all, dst_all, rows_v,
              acc_sh, gsem0, gsem1, ssem0, ssem1):
    cid = lax.axis_index("c")
    sid = lax.axis_index("s")

    pltpu.sync_copy(src_hbm.at[sid], src_all)
    pltpu.sync_copy(dst_hbm.at[sid], dst_all)

    gsems = (gsem0, gsem1)
    ssems = (ssem0, ssem1)

    for c in range(4):
        @pl.when(cid == (c // 2))
        def _():
            r0 = sid * ACC_RPT
            pltpu.sync_copy(h_hbm.at[c].at[pl.ds(r0, ACC_RPT)],
                            acc_sh.at[pl.ds(r0, ACC_RPT)])
        plsc.subcore_barrier()

        @pl.when(cid == (c // 2))
        def _():
            pltpu.async_copy(h_hbm.at[c].at[src_all.at[pl.ds(0, KB)]],
                             rows_v.at[0], gsem0)

            def group(gg, _):
                for b in range(2):
                    j = 2 * gg + b

                    @pl.when(j >= 1)
                    def _():
                        pltpu.make_async_copy(h_hbm.at[c].at[pl.ds(0, KB)],
                                              rows_v.at[1 - b],
                                              ssems[1 - b]).wait()

                    @pl.when(j < NBAT - 1)
                    def _():
                        pltpu.async_copy(
                            h_hbm.at[c].at[src_all.at[pl.ds((j + 1) * KB,
                                                            KB)]],
                            rows_v.at[1 - b], gsems[1 - b])

                    pltpu.make_async_copy(h_hbm.at[c].at[pl.ds(0, KB)],
                                          rows_v.at[b], gsems[b]).wait()
                    pltpu.async_copy(rows_v.at[b], acc_sh.at[dst_all.at[j]],
                                     ssems[b], add=True)
                return 0
            lax.fori_loop(0, NBAT // 2, group, 0)
            pltpu.make_async_copy(h_hbm.at[c].at[pl.ds(0, KB)],
                                  rows_v.at[(NBAT - 1) % 2],
                                  ssems[(NBAT - 1) % 2]).wait()
        plsc.subcore_barrier()

        @pl.when(cid == (c // 2))
        def _():
            r0 = sid * ACC_RPT
            pltpu.sync_copy(acc_sh.at[pl.ds(r0, ACC_RPT)],
                            out_hbm.at[c].at[pl.ds(r0, ACC_RPT)])
        plsc.subcore_barrier()


@functools.lru_cache(maxsize=None)
def _sc_kernels():
    mesh = plsc.VectorSubcoreMesh(core_axis_name="c", subcore_axis_name="s")
    deg = functools.partial(
        pl.kernel,
        mesh=mesh,
        out_type=jax.ShapeDtypeStruct((NC, N_PAD, _DEG_W), jnp.float32),
        scratch_types=[
            pltpu.VMEM((_DEG_NB, 128), jnp.int32),
            pltpu.VMEM((128, _DEG_W), jnp.float32),
            pltpu.VMEM((128, _DEG_W), jnp.float32),
            pltpu.VMEM_SHARED((N_PAD, _DEG_W), jnp.float32),
            pltpu.SemaphoreType.DMA,
        ],
    )(_deg_body)
    agg = functools.partial(
        pl.kernel,
        mesh=mesh,
        out_type=jax.ShapeDtypeStruct((4, N_PAD, 128), jnp.float32),
        scratch_types=[
            pltpu.VMEM((NBAT * KB,), jnp.int32),
            pltpu.VMEM((NBAT, KB), jnp.int32),
            pltpu.VMEM((2, KB, 128), jnp.float32),
            pltpu.VMEM_SHARED((ACC_ROWS, 128), jnp.float32),
        ] + [pltpu.SemaphoreType.DMA] * 4,
    )(_agg_body)
    return deg, agg


def _deg_kernel(dst):
    const = jnp.stack([jnp.ones((128, _DEG_W), jnp.float32),
                       jnp.zeros((128, _DEG_W), jnp.float32)])
    return _sc_kernels()[0](dst, const)


def _agg_kernel(h, src3, dst3):
    return _sc_kernels()[1](h, src3, dst3)


def _dinv(deg):
    return jnp.where(deg > 0.0, lax.rsqrt(deg), 0.0)


def _dot(a, b):
    return jnp.dot(a, b, preferred_element_type=jnp.float32,
                   precision=lax.Precision.HIGHEST)


def _tc1_body(x_ref, deg_ref, w_ref, out_ref):
    di = _dinv(deg_ref[...])
    h = _dot(x_ref[...], w_ref[...]) * di
    for c in range(4):
        out_ref[c] = h[:, c * 128:(c + 1) * 128]


def _tcmid_body(s_ref, deg_ref, al_ref, be_ref, w_ref, out_ref):
    di = _dinv(deg_ref[...])
    xb = jnp.concatenate([s_ref[c] for c in range(4)], axis=1)
    a = jnp.maximum(al_ref[...] * (xb * di) + be_ref[...], 0.0)
    h = _dot(a, w_ref[...]) * di
    for c in range(4):
        out_ref[c] = h[:, c * 128:(c + 1) * 128]


def _tcfinal_body(s_ref, deg_ref, al_ref, be_ref, brow_ref, bcol_ref,
                  bsmem_ref, ef_ref, we1_ref, bexp1_ref, we2_ref, bexp2_ref,
                  wf1_ref, bf1_ref, wf2_ref, bf2_ref,
                  out_ref, ge_ref, ee_ref,
                  mean_acc, max_acc, cnt_acc):
    i = pl.program_id(0)

    @pl.when(i == 0)
    def _():
        mean_acc[...] = jnp.zeros((B, H), jnp.float32)
        max_acc[...] = jnp.full((B, H), -3e38, jnp.float32)
        cnt_acc[...] = jnp.zeros((B, 1), jnp.float32)

    di = _dinv(deg_ref[...])
    xb = jnp.concatenate([s_ref[c] for c in range(4)], axis=1)
    h4 = jnp.maximum(al_ref[...] * (xb * di) + be_ref[...], 0.0)
    h4 = jnp.where(bcol_ref[...] < B, h4, 0.0)

    brow = brow_ref[0]
    seg = lax.broadcasted_iota(jnp.int32, (B, BLK), 0)
    mask = (brow == seg).astype(jnp.float32)
    mean_acc[...] += _dot(mask, h4)
    cnt_acc[...] += jnp.sum(mask, axis=1, keepdims=True)

    bcol = bcol_ref[...]
    b0 = bsmem_ref[0, 0, 0]
    b1 = jnp.minimum(bsmem_ref[0, 0, BLK - 1], B - 1)

    def seg_body(b, _):
        nm = bcol == b
        m = jnp.max(jnp.where(nm, h4, -3e38), axis=0, keepdims=True)
        max_acc[pl.ds(b, 1), :] = jnp.maximum(max_acc[pl.ds(b, 1), :], m)
        return 0
    lax.fori_loop(b0, b1 + 1, seg_body, 0)

    @pl.when(i == NBLK - 1)
    def _():
        cnt = cnt_acc[...]
        mean = mean_acc[...] / jnp.maximum(cnt, 1.0)
        mx = jnp.where(cnt > 0.0, max_acc[...], 0.0)
        ge_ref[:, 0:H] = mean
        ge_ref[:, H:2 * H] = mx
        ee = _dot(jnp.maximum(_dot(ef_ref[...], we1_ref[...])
                              + bexp1_ref[...], 0.0),
                  we2_ref[...]) + bexp2_ref[...]
        ee_ref[...] = ee
        cat = jnp.concatenate([mean, mx, ee], axis=1)
        fc = jnp.maximum(_dot(cat, wf1_ref[...]) + bf1_ref[...], 0.0)
        out_ref[...] = _dot(fc, wf2_ref[...]) + bf2_ref[...]


def _tc1(x_pad, deg_col, W1):
    return pl.pallas_call(
        _tc1_body,
        grid=(NBLK,),
        in_specs=[
            pl.BlockSpec((BLK, D_IN), lambda i: (i, 0)),
            pl.BlockSpec((BLK, 1), lambda i: (i, 0)),
            pl.BlockSpec((D_IN, H), lambda i: (0, 0)),
        ],
        out_specs=pl.BlockSpec((4, BLK, 128), lambda i: (0, i, 0)),
        out_shape=jax.ShapeDtypeStruct((4, N_PAD, 128), jnp.float32),
    )(x_pad, deg_col, W1)


def _tcmid(s, deg_col, al, be, W):
    return pl.pallas_call(
        _tcmid_body,
        grid=(NBLK,),
        in_specs=[
            pl.BlockSpec((4, BLK, 128), lambda i: (0, i, 0)),
            pl.BlockSpec((BLK, 1), lambda i: (i, 0)),
            pl.BlockSpec((1, H), lambda i: (0, 0)),
            pl.BlockSpec((1, H), lambda i: (0, 0)),
            pl.BlockSpec((H, H), lambda i: (0, 0)),
        ],
        out_specs=pl.BlockSpec((4, BLK, 128), lambda i: (0, i, 0)),
        out_shape=jax.ShapeDtypeStruct((4, N_PAD, 128), jnp.float32),
    )(s, deg_col, al, be, W)


def _tcfinal(s, deg_col, al, be, brow, bcol, bsmem, ef,
             We1, bexp1, We2, bexp2, Wf1, bf1, Wf2, bf2):
    return pl.pallas_call(
        _tcfinal_body,
        grid=(NBLK,),
        in_specs=[
            pl.BlockSpec((4, BLK, 128), lambda i: (0, i, 0)),
            pl.BlockSpec((BLK, 1), lambda i: (i, 0)),
            pl.BlockSpec((1, H), lambda i: (0, 0)),
            pl.BlockSpec((1, H), lambda i: (0, 0)),
            pl.BlockSpec((1, 1, BLK), lambda i: (i, 0, 0)),
            pl.BlockSpec((BLK, 1), lambda i: (i, 0)),
            pl.BlockSpec((1, 1, BLK), lambda i: (i, 0, 0),
                         memory_space=pltpu.SMEM),
            pl.BlockSpec((B, EXP_D), lambda i: (0, 0)),
            pl.BlockSpec((EXP_D, H), lambda i: (0, 0)),
            pl.BlockSpec((1, H), lambda i: (0, 0)),
            pl.BlockSpec((H, H), lambda i: (0, 0)),
            pl.BlockSpec((1, H), lambda i: (0, 0)),
            pl.BlockSpec((3 * H, 256), lambda i: (0, 0)),
            pl.BlockSpec((1, 256), lambda i: (0, 0)),
            pl.BlockSpec((256, 1), lambda i: (0, 0)),
            pl.BlockSpec((1, 1), lambda i: (0, 0)),
        ],
        out_specs=[
            pl.BlockSpec((B, 1), lambda i: (0, 0)),
            pl.BlockSpec((B, 2 * H), lambda i: (0, 0)),
            pl.BlockSpec((B, H), lambda i: (0, 0)),
        ],
        out_shape=[
            jax.ShapeDtypeStruct((B, 1), jnp.float32),
            jax.ShapeDtypeStruct((B, 2 * H), jnp.float32),
            jax.ShapeDtypeStruct((B, H), jnp.float32),
        ],
        scratch_shapes=[
            pltpu.VMEM((B, H), jnp.float32),
            pltpu.VMEM((B, H), jnp.float32),
            pltpu.VMEM((B, 1), jnp.float32),
        ],
    )(s, deg_col, al, be, brow, bcol, bsmem, ef,
      We1, bexp1, We2, bexp2, Wf1, bf1, Wf2, bf2)


def kernel(x, edge_index, batch, experimental_feat,
           W1, b1, g1, be1, W2, b2, g2, be2, W3, b3, g3, be3,
           W4, b4, g4, be4, We1, bexp1, We2, bexp2, Wf1, bf1, Wf2, bf2):
    x_pad = jnp.pad(x, ((0, N_PAD - N), (0, 0)))
    src3 = jnp.pad(edge_index[0], (0, E_PAD - E),
                   constant_values=0).reshape(NS, NBAT * KB)
    dst_pad = jnp.pad(edge_index[1], (0, E_PAD - E), constant_values=DUMMY)
    dst3 = dst_pad.reshape(NS, NBAT, KB)
    dst_deg = jnp.pad(edge_index[1], (0, E_PAD_DEG - E),
                      constant_values=DUMMY).reshape(NC, NS, _DEG_NB, 128)
    batch_pad = jnp.pad(batch, (0, N_PAD - N), constant_values=B)
    brow = batch_pad.reshape(NBLK, 1, BLK)
    bcol = batch_pad.reshape(N_PAD, 1)

    def fold(g, bconv, be):
        al = (g * _BN_C).reshape(1, H)
        return al, (al * bconv.reshape(1, H) + be.reshape(1, H))

    al1, bt1 = fold(g1, b1, be1)
    al2, bt2 = fold(g2, b2, be2)
    al3, bt3 = fold(g3, b3, be3)
    al4, bt4 = fold(g4, b4, be4)

    deg_parts = _deg_kernel(dst_deg)
    deg_col = (deg_parts[0, :, 0] + deg_parts[1, :, 0] + 1.0).reshape(N_PAD, 1)

    h = _tc1(x_pad, deg_col, W1)
    s = _agg_kernel(h, src3, dst3)
    h = _tcmid(s, deg_col, al1, bt1, W2)
    s = _agg_kernel(h, src3, dst3)
    h = _tcmid(s, deg_col, al2, bt2, W3)
    s = _agg_kernel(h, src3, dst3)
    h = _tcmid(s, deg_col, al3, bt3, W4)
    s = _agg_kernel(h, src3, dst3)

    out, graph_emb, exp_emb = _tcfinal(
        s, deg_col, al4, bt4, brow, bcol, brow, experimental_feat,
        We1, bexp1.reshape(1, H), We2, bexp2.reshape(1, H),
        Wf1, bf1.reshape(1, 256), Wf2, bf2.reshape(1, 1))
    return (out, graph_emb, exp_emb)
